# Initial kernel scaffold; baseline (speedup 1.0000x reference)
import jax
import jax.numpy as jnp
from jax.experimental import pallas as pl


def kernel(x, token_positions, W_QKV, W_O, qk_scale):
    raise NotImplementedError("write your pallas kernel here")


# speedup vs baseline: 2.5665x; 2.5665x over previous
import jax
import jax.numpy as jnp
from jax.experimental import pallas as pl
from jax.experimental.pallas import tpu as pltpu

D_MODEL = 1024
NUM_HEADS = 16
D_K = 64
THETA = 10000.0
EPS = 1e-8
BQ = 256


def _matmul_kernel(x_ref, w_ref, o_ref):
    o_ref[...] = jnp.dot(x_ref[...], w_ref[...],
                         preferred_element_type=jnp.float32)


def _matmul(x, w, bn):
    m, k = x.shape
    _, n = w.shape
    return pl.pallas_call(
        _matmul_kernel,
        grid=(n // bn,),
        in_specs=[
            pl.BlockSpec((m, k), lambda j: (0, 0)),
            pl.BlockSpec((k, bn), lambda j: (0, j)),
        ],
        out_specs=pl.BlockSpec((m, bn), lambda j: (0, j)),
        out_shape=jax.ShapeDtypeStruct((m, n), jnp.float32),
        compiler_params=pltpu.CompilerParams(
            dimension_semantics=("parallel",),
            vmem_limit_bytes=100 * 1024 * 1024,
        ),
    )(x, w)


def _attn_kernel(scale_ref, q_ref, k_ref, v_ref, cos_ref, sin_ref, o_ref,
                 qn_ref, kn_ref):
    hp = pl.program_id(0)
    seq = q_ref.shape[1]
    cos = cos_ref[...]
    sin = sin_ref[...]

    def rope_norm(xb):
        sw = jnp.concatenate(
            [xb[:, 32:64], xb[:, 0:32], xb[:, 96:128], xb[:, 64:96]], axis=1)
        r = cos * xb + sin * sw

        def norm_half(u):
            ss = jnp.sum(u * u, axis=1, keepdims=True)
            return u / (jnp.sqrt(ss) + EPS)

        return jnp.concatenate([norm_half(r[:, :64]), norm_half(r[:, 64:])],
                               axis=1)

    kn_ref[...] = rope_norm(k_ref[0])
    qn = rope_norm(q_ref[0])
    g0 = scale_ref[2 * hp]
    g1 = scale_ref[2 * hp + 1]
    lane = jax.lax.broadcasted_iota(jnp.int32, (seq, 2 * D_K), 1)
    qn_ref[...] = qn * jnp.where(lane < D_K, g0, g1)

    for qi in range(seq // BQ):
        nk = (qi + 1) * BQ
        for s in range(2):
            qb = qn_ref[qi * BQ:(qi + 1) * BQ, s * D_K:(s + 1) * D_K]
            kb = kn_ref[0:nk, s * D_K:(s + 1) * D_K]
            sc = jax.lax.dot_general(
                qb, kb, (((1,), (1,)), ((), ())),
                preferred_element_type=jnp.float32)
            cols = jax.lax.broadcasted_iota(jnp.int32, (BQ, nk), 1)
            rows = jax.lax.broadcasted_iota(jnp.int32, (BQ, nk), 0) + qi * BQ
            sc = jnp.where(cols <= rows, sc, -jnp.inf)
            m = jnp.max(sc, axis=-1, keepdims=True)
            e = jnp.exp(sc - m)
            denom = jnp.sum(e, axis=-1, keepdims=True)
            vb = v_ref[0, 0:nk, s * D_K:(s + 1) * D_K]
            ov = jax.lax.dot_general(
                e, vb, (((1,), (0,)), ((), ())),
                preferred_element_type=jnp.float32)
            o_ref[0, qi * BQ:(qi + 1) * BQ, s * D_K:(s + 1) * D_K] = (
                ov * (1.0 / denom))


def _attention(qkv, cos, sin, qk_scale):
    b, n, _ = qkv.shape
    hpairs = NUM_HEADS // 2
    return pl.pallas_call(
        _attn_kernel,
        grid=(hpairs, b),
        in_specs=[
            pl.BlockSpec(memory_space=pltpu.SMEM),
            pl.BlockSpec((1, n, 128), lambda hp, bi: (bi, 0, hp)),
            pl.BlockSpec((1, n, 128), lambda hp, bi: (bi, 0, hpairs + hp)),
            pl.BlockSpec((1, n, 128), lambda hp, bi: (bi, 0, 2 * hpairs + hp)),
            pl.BlockSpec((n, 128), lambda hp, bi: (0, 0)),
            pl.BlockSpec((n, 128), lambda hp, bi: (0, 0)),
        ],
        out_specs=pl.BlockSpec((1, n, 128), lambda hp, bi: (bi, 0, hp)),
        out_shape=jax.ShapeDtypeStruct((b, n, D_MODEL), jnp.float32),
        scratch_shapes=[
            pltpu.VMEM((n, 128), jnp.float32),
            pltpu.VMEM((n, 128), jnp.float32),
        ],
        compiler_params=pltpu.CompilerParams(
            dimension_semantics=("parallel", "parallel"),
            vmem_limit_bytes=100 * 1024 * 1024,
        ),
    )(qk_scale, qkv, qkv, qkv, cos, sin)


def kernel(x, token_positions, W_QKV, W_O, qk_scale):
    b, n, d = x.shape

    def permute_half_split(w):
        return (w.reshape(NUM_HEADS, D_K // 2, 2, d)
                 .transpose(0, 2, 1, 3).reshape(d, d))

    w_q = permute_half_split(W_QKV[:D_MODEL])
    w_k = permute_half_split(W_QKV[D_MODEL:2 * D_MODEL])
    w_v = W_QKV[2 * D_MODEL:]
    w_all_t = jnp.concatenate([w_q, w_k, w_v], axis=0).T

    qkv = _matmul(x.reshape(b * n, d), w_all_t, 256).reshape(b, n, 3 * d)

    pos = token_positions.astype(jnp.float32)
    inv_theta = THETA ** (-(2.0 * jnp.arange(D_K // 2, dtype=jnp.float32))
                          / D_K)
    ang = pos[:, None] * inv_theta[None, :]
    c32, s32 = jnp.cos(ang), jnp.sin(ang)
    cos = jnp.tile(jnp.concatenate([c32, c32], axis=1), (1, 2))
    sin = jnp.tile(jnp.concatenate([-s32, s32], axis=1), (1, 2))

    attn = _attention(qkv, cos, sin, qk_scale)
    out = _matmul(attn.reshape(b * n, d), W_O.T, 256)
    return out.reshape(b, n, d)


# speedup vs baseline: 3.5367x; 1.3780x over previous
import jax
import jax.numpy as jnp
from jax.experimental import pallas as pl
from jax.experimental.pallas import tpu as pltpu

D_MODEL = 1024
NUM_HEADS = 16
D_K = 64
THETA = 10000.0
EPS = 1e-8
BQ = 256
LOG2E = 1.4426950408889634


def _matmul_kernel(x_ref, w_ref, o_ref):
    o_ref[...] = jnp.dot(x_ref[...], w_ref[...],
                         preferred_element_type=jnp.float32)


def _matmul_kernel_bf16(x_ref, w_ref, o_ref):
    o_ref[...] = jnp.dot(x_ref[...], w_ref[...].astype(jnp.bfloat16),
                         preferred_element_type=jnp.float32)


def _matmul(x, w, bn, body=_matmul_kernel):
    m, k = x.shape
    _, n = w.shape
    return pl.pallas_call(
        body,
        grid=(n // bn,),
        in_specs=[
            pl.BlockSpec((m, k), lambda j: (0, 0)),
            pl.BlockSpec((k, bn), lambda j: (0, j)),
        ],
        out_specs=pl.BlockSpec((m, bn), lambda j: (0, j)),
        out_shape=jax.ShapeDtypeStruct((m, n), jnp.float32),
        compiler_params=pltpu.CompilerParams(
            dimension_semantics=("parallel",),
            vmem_limit_bytes=100 * 1024 * 1024,
        ),
    )(x, w)


def _attn_kernel(q_ref, k_ref, v_ref, cos_ref, sin_ref, g_ref, o_ref,
                 qn_ref, kn_ref, vn_ref):
    seq = q_ref.shape[1]
    cos = cos_ref[...]
    sin = sin_ref[...]

    def rope_norm(xb):
        sw = jnp.concatenate(
            [xb[:, 32:64], xb[:, 0:32], xb[:, 96:128], xb[:, 64:96]], axis=1)
        r = cos * xb + sin * sw

        def norm_half(u):
            ss = jnp.sum(u * u, axis=1, keepdims=True)
            return u / (jnp.sqrt(ss) + EPS)

        return jnp.concatenate([norm_half(r[:, :64]), norm_half(r[:, 64:])],
                               axis=1)

    kn_ref[...] = rope_norm(k_ref[0])
    qn_ref[...] = rope_norm(q_ref[0]) * g_ref[0]
    vn_ref[...] = v_ref[0].astype(jnp.bfloat16)

    rows_d = jax.lax.broadcasted_iota(jnp.int32, (BQ, BQ), 0)
    cols_d = jax.lax.broadcasted_iota(jnp.int32, (BQ, BQ), 1)
    dmask = cols_d <= rows_d

    for qi in range(seq // BQ):
        base = qi * BQ
        for s in range(2):
            lo, hi = s * D_K, (s + 1) * D_K
            qb = qn_ref[base:base + BQ, lo:hi]
            sc_d = jax.lax.dot_general(
                qb, kn_ref[base:base + BQ, lo:hi], (((1,), (1,)), ((), ())),
                preferred_element_type=jnp.float32)
            e_d = jnp.where(dmask, jnp.exp2(sc_d), 0.0)
            denom = jnp.sum(e_d, axis=-1, keepdims=True)
            acc = jax.lax.dot_general(
                e_d.astype(jnp.bfloat16), vn_ref[base:base + BQ, lo:hi],
                (((1,), (0,)), ((), ())),
                preferred_element_type=jnp.float32)
            if qi > 0:
                sc_h = jax.lax.dot_general(
                    qb, kn_ref[0:base, lo:hi], (((1,), (1,)), ((), ())),
                    preferred_element_type=jnp.float32)
                e_h = jnp.exp2(sc_h)
                denom = denom + jnp.sum(e_h, axis=-1, keepdims=True)
                acc = acc + jax.lax.dot_general(
                    e_h.astype(jnp.bfloat16), vn_ref[0:base, lo:hi],
                    (((1,), (0,)), ((), ())),
                    preferred_element_type=jnp.float32)
            o_ref[0, base:base + BQ, lo:hi] = (
                acc * (1.0 / denom)).astype(jnp.bfloat16)


def _attention(qkv, cos, sin, garr):
    b, n, _ = qkv.shape
    hpairs = NUM_HEADS // 2
    return pl.pallas_call(
        _attn_kernel,
        grid=(hpairs, b),
        in_specs=[
            pl.BlockSpec((1, n, 128), lambda hp, bi: (bi, 0, hp)),
            pl.BlockSpec((1, n, 128), lambda hp, bi: (bi, 0, hpairs + hp)),
            pl.BlockSpec((1, n, 128), lambda hp, bi: (bi, 0, 2 * hpairs + hp)),
            pl.BlockSpec((n, 128), lambda hp, bi: (0, 0)),
            pl.BlockSpec((n, 128), lambda hp, bi: (0, 0)),
            pl.BlockSpec((1, 1, 128), lambda hp, bi: (hp, 0, 0)),
        ],
        out_specs=pl.BlockSpec((1, n, 128), lambda hp, bi: (bi, 0, hp)),
        out_shape=jax.ShapeDtypeStruct((b, n, D_MODEL), jnp.bfloat16),
        scratch_shapes=[
            pltpu.VMEM((n, 128), jnp.float32),
            pltpu.VMEM((n, 128), jnp.float32),
            pltpu.VMEM((n, 128), jnp.bfloat16),
        ],
        compiler_params=pltpu.CompilerParams(
            dimension_semantics=("parallel", "parallel"),
            vmem_limit_bytes=100 * 1024 * 1024,
        ),
    )(qkv, qkv, qkv, cos, sin, garr)


def kernel(x, token_positions, W_QKV, W_O, qk_scale):
    b, n, d = x.shape

    def permute_half_split(w):
        return (w.reshape(NUM_HEADS, D_K // 2, 2, d)
                 .transpose(0, 2, 1, 3).reshape(d, d))

    w_q = permute_half_split(W_QKV[:D_MODEL])
    w_k = permute_half_split(W_QKV[D_MODEL:2 * D_MODEL])
    w_v = W_QKV[2 * D_MODEL:]
    w_all_t = jnp.concatenate([w_q, w_k, w_v], axis=0).T

    qkv = _matmul(x.reshape(b * n, d), w_all_t, 256).reshape(b, n, 3 * d)

    pos = token_positions.astype(jnp.float32)
    inv_theta = THETA ** (-(2.0 * jnp.arange(D_K // 2, dtype=jnp.float32))
                          / D_K)
    ang = pos[:, None] * inv_theta[None, :]
    c32, s32 = jnp.cos(ang), jnp.sin(ang)
    cos = jnp.tile(jnp.concatenate([c32, c32], axis=1), (1, 2))
    sin = jnp.tile(jnp.concatenate([-s32, s32], axis=1), (1, 2))

    garr = jnp.repeat(qk_scale * LOG2E, D_K).reshape(NUM_HEADS // 2, 1, 128)

    attn = _attention(qkv, cos, sin, garr)
    out = _matmul(attn.reshape(b * n, d), W_O.T, 256,
                  body=_matmul_kernel_bf16)
    return out.reshape(b, n, d)
